# baseline (speedup 1.0000x reference)
import functools

import jax
import jax.numpy as jnp
from jax import lax
from jax.experimental import pallas as pl
from jax.experimental.pallas import tpu as pltpu
from jax.experimental.pallas import tpu_sc as plsc

EMB_DIM = 256
CODEBOOK = 8192
TOKENS = 8192
TB = 1024
CB = 1024
NT = TOKENS // TB
NCB = CODEBOOK // CB
LOSS_SCALE = 1.25 / (TOKENS * EMB_DIM)


def _stage1_body(x_ref, e_ref, idx_ref, loss_ref, enc_ref,
                 minv, mini, idxv, acc):
    i = pl.program_id(0)
    jp = pl.program_id(1)

    @pl.when(jnp.logical_and(i == 0, jp == 0))
    def _():
        acc[0] = jnp.float32(0.0)

    @pl.when(jp == 0)
    def _():
        minv[...] = jnp.full((TB, 128), jnp.inf, jnp.float32)
        mini[...] = jnp.zeros((TB, 128), jnp.int32)

    @pl.when(jp < NCB)
    def _():
        j = jp
        xb = x_ref[...]
        eb = e_ref[...]
        s = lax.dot_general(xb, eb, (((1,), (0,)), ((), ())),
                            preferred_element_type=jnp.float32)
        a = jnp.sum(xb * xb, axis=1, keepdims=True)
        b = jnp.sum(eb * eb, axis=0)
        m = minv[...]
        ii = mini[...]
        for k in range(CB // 128):
            sk = lax.slice(s, (0, k * 128), (TB, (k + 1) * 128))
            bk = lax.slice(b, (k * 128,), ((k + 1) * 128,))
            dk = (a + bk) - 2.0 * sk
            cond = dk < m
            m = jnp.where(cond, dk, m)
            ii = jnp.where(cond, jnp.int32(j * (CB // 128) + k), ii)
        minv[...] = m
        mini[...] = ii

        @pl.when(j == NCB - 1)
        def _():
            lane = lax.broadcasted_iota(jnp.int32, (TB, 128), 1)
            gmin = jnp.min(m, axis=1)
            cand = jnp.where(m == gmin[:, None], ii * 128 + lane,
                             jnp.int32(0x7FFFFFFF))
            win = jnp.min(cand, axis=1)
            idxv[...] = win
            idx_ref[...] = win
            acc[0] = acc[0] + jnp.sum(gmin)

    @pl.when(jp >= NCB)
    def _():
        jj = jp - NCB
        cols = lax.broadcasted_iota(jnp.int32, (TB, CB), 1) + jj * CB
        enc_ref[...] = (idxv[...][:, None] == cols).astype(jnp.float32)

    @pl.when(jnp.logical_and(i == NT - 1, jp == 2 * NCB - 1))
    def _():
        loss_ref[0, 0] = acc[0] * LOSS_SCALE


def _argmin_onehot_loss(x, emb):
    return pl.pallas_call(
        _stage1_body,
        grid=(NT, 2 * NCB),
        in_specs=[
            pl.BlockSpec((TB, EMB_DIM), lambda i, jp: (i, 0)),
            pl.BlockSpec((EMB_DIM, CB),
                         lambda i, jp: (0, jnp.minimum(jp, NCB - 1))),
        ],
        out_specs=[
            pl.BlockSpec((TB,), lambda i, jp: (i,)),
            pl.BlockSpec(memory_space=pltpu.SMEM),
            pl.BlockSpec((TB, CB),
                         lambda i, jp: (i, jnp.maximum(jp - NCB, 0))),
        ],
        out_shape=[
            jax.ShapeDtypeStruct((TOKENS,), jnp.int32),
            jax.ShapeDtypeStruct((1, 1), jnp.float32),
            jax.ShapeDtypeStruct((TOKENS, CODEBOOK), jnp.float32),
        ],
        scratch_shapes=[
            pltpu.VMEM((TB, 128), jnp.float32),
            pltpu.VMEM((TB, 128), jnp.int32),
            pltpu.VMEM((TB,), jnp.int32),
            pltpu.SMEM((1,), jnp.float32),
        ],
        compiler_params=pltpu.CompilerParams(
            dimension_semantics=("arbitrary", "arbitrary")),
    )(x, emb)


def _sc_gather(table, idx):
    info = plsc.get_sparse_core_info()
    nc, ns = info.num_cores, info.num_subcores
    nw = nc * ns
    b_per_w = TOKENS // nw
    mesh = plsc.VectorSubcoreMesh(core_axis_name="c", subcore_axis_name="s")

    @functools.partial(
        pl.kernel, mesh=mesh,
        out_type=jax.ShapeDtypeStruct((TOKENS, EMB_DIM), jnp.float32),
        scratch_types=[
            pltpu.VMEM((b_per_w,), jnp.int32),
            pltpu.VMEM((b_per_w, EMB_DIM), jnp.float32),
            pltpu.SemaphoreType.DMA,
        ],
    )
    def gather_k(table_hbm, idx_hbm, out_hbm, idx_v, rows_v, sem):
        wid = lax.axis_index("s") * nc + lax.axis_index("c")
        base = wid * b_per_w
        pltpu.sync_copy(idx_hbm.at[pl.ds(base, b_per_w)], idx_v)
        pltpu.async_copy(table_hbm.at[idx_v], rows_v, sem).wait()
        pltpu.sync_copy(rows_v, out_hbm.at[pl.ds(base, b_per_w)])

    return gather_k(table, idx)


def kernel(inputs, embeddings):
    x = inputs.reshape(-1, EMB_DIM)
    idx, loss11, encodings = _argmin_onehot_loss(x, embeddings)
    emb_t = jnp.swapaxes(embeddings, 0, 1)
    quantized = _sc_gather(emb_t, idx)
    quantized_st = quantized.reshape(inputs.shape)
    encoding_indices = idx.reshape(inputs.shape[:-1])
    loss = loss11[0, 0]
    return quantized_st, encodings, encoding_indices, loss
